# baseline (device time: 12084 ns/iter reference)
import jax
import jax.numpy as jnp
from jax import lax
from jax.experimental import pallas as pl
from jax.experimental.pallas import tpu as pltpu

N_CHUNKS = 16


def kernel(x):
    m, n = x.shape
    chunk = m // N_CHUNKS
    assert m % (8 * N_CHUNKS) == 0

    def body(x_hbm, out_ref, xb_ref, acc_ref, send_ref, recv_ref,
             copy_sems, send_sem, recv_sem):
        my_x = lax.axis_index("x")
        my_y = lax.axis_index("y")
        peer = (1 - my_x, my_y)

        copies = []
        for k in range(N_CHUNKS):
            cp = pltpu.make_async_copy(
                x_hbm.at[pl.ds(k * chunk, chunk), :],
                xb_ref.at[pl.ds(k * chunk, chunk), :],
                copy_sems.at[k],
            )
            cp.start()
            copies.append(cp)

        barrier_sem = pltpu.get_barrier_semaphore()
        pl.semaphore_signal(
            barrier_sem, inc=1, device_id=peer,
            device_id_type=pl.DeviceIdType.MESH,
        )

        for k in range(N_CHUNKS):
            copies[k].wait()
            xb = xb_ref[pl.ds(k * chunk, chunk), :].reshape(chunk // 8, 8, n)
            cm = jnp.max(xb, axis=0)
            if k == 0:
                acc_ref[:, :] = cm
            else:
                acc_ref[:, :] = jnp.maximum(acc_ref[:, :], cm)

        send_ref[0, :] = jnp.max(acc_ref[:, :], axis=0)

        pl.semaphore_wait(barrier_sem, 1)
        rdma = pltpu.make_async_remote_copy(
            src_ref=send_ref,
            dst_ref=recv_ref,
            send_sem=send_sem,
            recv_sem=recv_sem,
            device_id=peer,
            device_id_type=pl.DeviceIdType.MESH,
        )
        rdma.start()
        rdma.wait()
        out_ref[:, :] = jnp.maximum(send_ref[:, :], recv_ref[:, :])

    return pl.pallas_call(
        body,
        out_shape=jax.ShapeDtypeStruct((1, n), x.dtype),
        in_specs=[pl.BlockSpec(memory_space=pl.ANY)],
        out_specs=pl.BlockSpec(memory_space=pltpu.VMEM),
        scratch_shapes=[
            pltpu.VMEM((m, n), x.dtype),
            pltpu.VMEM((8, n), x.dtype),
            pltpu.VMEM((1, n), x.dtype),
            pltpu.VMEM((1, n), x.dtype),
            pltpu.SemaphoreType.DMA((N_CHUNKS,)),
            pltpu.SemaphoreType.DMA,
            pltpu.SemaphoreType.DMA,
        ],
        compiler_params=pltpu.CompilerParams(collective_id=0),
    )(x)
